# Initial kernel scaffold; baseline (speedup 1.0000x reference)
import jax
import jax.numpy as jnp
from jax.experimental import pallas as pl


def kernel(x, token_ids, gate_up_proj, down_proj):
    raise NotImplementedError("write your pallas kernel here")


# speedup vs baseline: 2.4080x; 2.4080x over previous
import functools

import jax
import jax.numpy as jnp
from jax.experimental import pallas as pl
from jax.experimental.pallas import tpu as pltpu

T = 4096
H = 1024
E = 8
EI = 1024
V = 32000
R = 256
T_PAD = T + E * R
NT = T_PAD // R


def _tc_body(map_ref, x_ref, gu_ref, dn_ref, o_ref):
    xb = x_ref[...].astype(jnp.bfloat16)
    w = gu_ref[0].astype(jnp.bfloat16)
    gu = jnp.dot(xb, w, preferred_element_type=jnp.float32)
    gate = gu[:, :EI]
    up = gu[:, EI:]
    inter = (gate * jax.nn.sigmoid(gate) * up).astype(jnp.bfloat16)
    o_ref[...] = jnp.dot(inter, dn_ref[0].astype(jnp.bfloat16),
                         preferred_element_type=jnp.float32)


def _grouped_mlp(x_sorted, gate_up_proj, down_proj, tile_expert, *, interpret=False):
    grid_spec = pltpu.PrefetchScalarGridSpec(
        num_scalar_prefetch=1,
        grid=(NT,),
        in_specs=[
            pl.BlockSpec((R, H), lambda i, m: (i, 0)),
            pl.BlockSpec((1, H, 2 * EI), lambda i, m: (m[i], 0, 0)),
            pl.BlockSpec((1, EI, H), lambda i, m: (m[i], 0, 0)),
        ],
        out_specs=pl.BlockSpec((R, H), lambda i, m: (i, 0)),
    )
    return pl.pallas_call(
        _tc_body,
        grid_spec=grid_spec,
        out_shape=jax.ShapeDtypeStruct((T_PAD, H), jnp.float32),
        interpret=interpret,
    )(tile_expert, x_sorted, gate_up_proj, down_proj)


def _route_jnp(token_ids):
    ids = jnp.clip(token_ids.astype(jnp.int32), 0, V - 1)
    e = ids % E
    counts = jnp.bincount(e, length=E)
    aligned = ((counts + R - 1) // R) * R
    off = jnp.concatenate([jnp.zeros((1,), jnp.int32),
                           jnp.cumsum(aligned)[:-1].astype(jnp.int32)])
    start = jnp.concatenate([jnp.zeros((1,), jnp.int32),
                             jnp.cumsum(counts)[:-1].astype(jnp.int32)])
    order = jnp.argsort(e, stable=True)
    e_sorted = e[order]
    rank_sorted = jnp.arange(T, dtype=jnp.int32) - start[e_sorted]
    dst_sorted = off[e_sorted] + rank_sorted
    dst = jnp.zeros((T,), jnp.int32).at[order].set(dst_sorted)
    tile_starts = jnp.arange(NT, dtype=jnp.int32) * R
    tile_expert = jnp.clip(
        jnp.searchsorted(off, tile_starts, side="right").astype(jnp.int32) - 1,
        0, E - 1)
    return dst, tile_expert


def kernel(x, token_ids, gate_up_proj, down_proj):
    dst, tile_expert = _route_jnp(token_ids)
    x_sorted = jnp.zeros((T_PAD, H), jnp.float32).at[dst].set(x)
    y_sorted = _grouped_mlp(x_sorted, gate_up_proj, down_proj, tile_expert)
    return y_sorted[dst]
